# baseline (speedup 1.0000x reference)
import dataclasses
import math

import jax
import jax.numpy as jnp
import numpy as np
from jax import lax
from jax.experimental import pallas as pl
from jax.experimental.pallas import tpu as pltpu
from jax.experimental.pallas import tpu_sc as plsc

FEAT = 128
N_ROWS = 1000000
BATCH = 1024
KP1 = 512
NCE_T = 0.07
EPS = 1e-7

NC, NS = 2, 16
NW = NC * NS
TOTAL = BATCH * KP1
PER_W = TOTAL // NW
CHUNK = 128
CPW = PER_W // CHUNK
BPW = BATCH // NW
CPB = KP1 // CHUNK


def _sc_gather_dot(memory_v1, memory_v2, flat_idx, f_t, f_s):
    mesh = plsc.VectorSubcoreMesh(core_axis_name="c", subcore_axis_name="s")
    out = jax.ShapeDtypeStruct((TOTAL * 16,), jnp.float32)

    def body(m1_hbm, m2_hbm, idx_hbm, ft_hbm, fs_hbm, da_hbm, db_hbm,
             idx_v, ft_v, fs_v, r1, r2, dv1a, dv1b, dv2a, dv2b,
             g1a, g1b, g2a, g2b, s1a, s1b, s2a, s2b):
        dv1 = (dv1a, dv1b)
        dv2 = (dv2a, dv2b)
        wid = lax.axis_index("s") * NC + lax.axis_index("c")
        base = wid * PER_W
        gsem1 = (g1a, g1b)
        gsem2 = (g2a, g2b)
        ssem1 = (s1a, s1b)
        ssem2 = (s2a, s2b)

        pltpu.sync_copy(idx_hbm.at[pl.ds(base, PER_W)], idx_v)
        pltpu.sync_copy(ft_hbm.at[pl.ds(wid * BPW, BPW)], ft_v)
        pltpu.sync_copy(fs_hbm.at[pl.ds(wid * BPW, BPW)], fs_v)

        def start_gathers(ci, p):
            sl = pl.ds(ci * CHUNK, CHUNK)
            pltpu.async_copy(m1_hbm.at[idx_v.at[sl]], r1.at[p], gsem1[p])
            pltpu.async_copy(m2_hbm.at[idx_v.at[sl]], r2.at[p], gsem2[p])

        def process(ci, p):
            sl = pl.ds(ci * CHUNK, CHUNK)
            pltpu.make_async_copy(m1_hbm.at[idx_v.at[sl]], r1.at[p],
                                  gsem1[p]).wait()
            pltpu.make_async_copy(m2_hbm.at[idx_v.at[sl]], r2.at[p],
                                  gsem2[p]).wait()

            bl = ci // CPB
            ftc = [ft_v[bl, pl.ds(16 * c, 16)] for c in range(8)]
            fsc = [fs_v[bl, pl.ds(16 * c, 16)] for c in range(8)]
            r1p, r2p = r1.at[p], r2.at[p]
            dv1p, dv2p = dv1[p], dv2[p]

            @pl.loop(0, CHUNK)
            def _(j):
                a1 = r1p[j, pl.ds(0, 16)] * ftc[0]
                a2 = r2p[j, pl.ds(0, 16)] * fsc[0]
                for c in range(1, 8):
                    a1 = a1 + r1p[j, pl.ds(16 * c, 16)] * ftc[c]
                    a2 = a2 + r2p[j, pl.ds(16 * c, 16)] * fsc[c]
                dv1p[pl.ds(j * 16, 16)] = a1
                dv2p[pl.ds(j * 16, 16)] = a2

            gsl = pl.ds((base + ci * CHUNK) * 16, CHUNK * 16)

            @pl.when(ci >= 2)
            def _():
                pltpu.make_async_copy(dv1p, da_hbm.at[gsl], ssem1[p]).wait()
                pltpu.make_async_copy(dv2p, db_hbm.at[gsl], ssem2[p]).wait()

            pltpu.async_copy(dv1p, da_hbm.at[gsl], ssem1[p])
            pltpu.async_copy(dv2p, db_hbm.at[gsl], ssem2[p])

            @pl.when(ci + 2 < CPW)
            def _():
                start_gathers(ci + 2, p)

        start_gathers(0, 0)
        start_gathers(1, 1)

        @pl.loop(0, CPW, step=2)
        def _(i):
            process(i, 0)
            process(i + 1, 1)

        gsl0 = pl.ds(base * 16, CHUNK * 16)
        for p in range(2):
            pltpu.make_async_copy(dv1[p], da_hbm.at[gsl0], ssem1[p]).wait()
            pltpu.make_async_copy(dv2[p], db_hbm.at[gsl0], ssem2[p]).wait()

    cp = pltpu.CompilerParams()
    if "needs_layout_passes" in pltpu.CompilerParams.__dataclass_fields__:
        cp = dataclasses.replace(cp, needs_layout_passes=False)
    k = pl.kernel(
        body,
        out_type=(out, out),
        mesh=mesh,
        compiler_params=cp,
        scratch_types=[
            pltpu.VMEM((PER_W,), jnp.int32),
            pltpu.VMEM((BPW, FEAT), jnp.float32),
            pltpu.VMEM((BPW, FEAT), jnp.float32),
            pltpu.VMEM((2, CHUNK, FEAT), jnp.float32),
            pltpu.VMEM((2, CHUNK, FEAT), jnp.float32),
            pltpu.VMEM((CHUNK * 16,), jnp.float32),
            pltpu.VMEM((CHUNK * 16,), jnp.float32),
            pltpu.VMEM((CHUNK * 16,), jnp.float32),
            pltpu.VMEM((CHUNK * 16,), jnp.float32),
        ] + [pltpu.SemaphoreType.DMA] * 8,
    )
    return k(memory_v1, memory_v2, flat_idx, f_t, f_s)


def _tc_tail(pa, pb):
    xa = pa.reshape(TOTAL // 8, 128)
    xb = pb.reshape(TOTAL // 8, 128)
    blk = 2048
    nblk = TOTAL // 8 // blk
    zscale = float(N_ROWS) / float(TOTAL)
    mpn = (KP1 - 1) / N_ROWS
    c = mpn + EPS
    log_mpn = math.log(mpn)
    nall = blk * 8
    npos = blk // 64
    ngarb = blk * 120

    li = np.arange(128)[:, None]
    lj = np.arange(128)[None, :]
    seg_mat = jnp.asarray((lj == (li // 16) * 16) * 1.0, jnp.float32)

    def body(xa_ref, xb_ref, m_ref, out_ref, acc):
        ph = pl.program_id(0)
        rows = lax.broadcasted_iota(jnp.int32, (blk, 128), 0)
        lanes = lax.broadcasted_iota(jnp.int32, (blk, 128), 1)
        seg = lanes % 16 == 0
        posm = (lanes == 0) & (rows % 64 == 0)
        dn = (((1,), (0,)), ((), ()))

        invt = jnp.float32(1.0 / NCE_T)
        da = lax.dot_general(xa_ref[...], m_ref[...], dn,
                             preferred_element_type=jnp.float32) * invt
        db = lax.dot_general(xb_ref[...], m_ref[...], dn,
                             preferred_element_type=jnp.float32) * invt
        ea = jnp.exp(da)
        eb = jnp.exp(db)

        @pl.when((ph == 0) & (pl.program_id(1) == 0))
        def _():
            acc[0] = 0.0
            acc[1] = 0.0
            out_ref[...] = jnp.zeros((1, 1), jnp.float32)

        @pl.when(ph == 0)
        def _():
            acc[0] += jnp.sum(ea) - float(ngarb)
            acc[1] += jnp.sum(eb) - float(ngarb)

        @pl.when(ph == 1)
        def _():
            zero = jnp.float32(0.0)
            ninf = jnp.float32(-jnp.inf)

            def side(d, e, z):
                o = e / z
                lgt = jnp.log(o + c)
                lg0 = log_mpn - lgt
                lg1 = jnp.where(o == 0.0, ninf, d - jnp.log(z)) - lgt
                return (jnp.sum(jnp.where(posm, lg1, zero))
                        + jnp.sum(jnp.where(seg, lg0, zero))
                        - jnp.sum(jnp.where(posm, lg0, zero)))

            la = side(da, ea, acc[0] * zscale)
            lb = side(db, eb, acc[1] * zscale)
            out_ref[...] += (-(la + lb) / BATCH).reshape(1, 1)

    return pl.pallas_call(
        body,
        grid=(2, nblk),
        in_specs=[pl.BlockSpec((blk, 128), lambda ph, j: (j, 0)),
                  pl.BlockSpec((blk, 128), lambda ph, j: (j, 0)),
                  pl.BlockSpec((128, 128), lambda ph, j: (0, 0))],
        out_specs=pl.BlockSpec((1, 1), lambda ph, j: (0, 0)),
        out_shape=jax.ShapeDtypeStruct((1, 1), jnp.float32),
        scratch_shapes=[pltpu.SMEM((2,), jnp.float32)],
    )(xa, xb, seg_mat)


def kernel(f_s, f_t, idx, contrast_idx, memory_v1, memory_v2):
    flat_idx = contrast_idx.reshape(-1)
    pa, pb = _sc_gather_dot(memory_v1, memory_v2, flat_idx, f_t, f_s)
    loss = _tc_tail(pa, pb)
    return loss.reshape(1)


# speedup vs baseline: 1.0523x; 1.0523x over previous
import dataclasses
import math

import jax
import jax.numpy as jnp
import numpy as np
from jax import lax
from jax.experimental import pallas as pl
from jax.experimental.pallas import tpu as pltpu
from jax.experimental.pallas import tpu_sc as plsc

FEAT = 128
N_ROWS = 1000000
BATCH = 1024
KP1 = 512
NCE_T = 0.07
EPS = 1e-7

NC, NS = 2, 16
NW = NC * NS
TOTAL = BATCH * KP1
PER_W = TOTAL // NW
CHUNK = 128
CPW = PER_W // CHUNK
BPW = BATCH // NW
CPB = KP1 // CHUNK


def _sc_gather_dot(memory_v1, memory_v2, flat_idx, f_t, f_s):
    mesh = plsc.VectorSubcoreMesh(core_axis_name="c", subcore_axis_name="s")
    out = jax.ShapeDtypeStruct((TOTAL * 16,), jnp.float32)

    def body(m1_hbm, m2_hbm, idx_hbm, ft_hbm, fs_hbm, da_hbm, db_hbm,
             idx_v, ft_v, fs_v, r1, r2, dv1a, dv1b, dv2a, dv2b,
             g1a, g1b, g2a, g2b, s1a, s1b, s2a, s2b):
        dv1 = (dv1a, dv1b)
        dv2 = (dv2a, dv2b)
        wid = lax.axis_index("s") * NC + lax.axis_index("c")
        base = wid * PER_W
        gsem1 = (g1a, g1b)
        gsem2 = (g2a, g2b)
        ssem1 = (s1a, s1b)
        ssem2 = (s2a, s2b)

        pltpu.sync_copy(idx_hbm.at[pl.ds(base, PER_W)], idx_v)
        pltpu.sync_copy(ft_hbm.at[pl.ds(wid * BPW, BPW)], ft_v)
        pltpu.sync_copy(fs_hbm.at[pl.ds(wid * BPW, BPW)], fs_v)

        def start_gathers(ci, p):
            sl = pl.ds(ci * CHUNK, CHUNK)
            pltpu.async_copy(m1_hbm.at[idx_v.at[sl]], r1.at[p], gsem1[p])
            pltpu.async_copy(m2_hbm.at[idx_v.at[sl]], r2.at[p], gsem2[p])

        def process(ci, p):
            sl = pl.ds(ci * CHUNK, CHUNK)
            pltpu.make_async_copy(m1_hbm.at[idx_v.at[sl]], r1.at[p],
                                  gsem1[p]).wait()
            pltpu.make_async_copy(m2_hbm.at[idx_v.at[sl]], r2.at[p],
                                  gsem2[p]).wait()

            bl = ci // CPB
            ftc = [ft_v[bl, pl.ds(16 * c, 16)] for c in range(8)]
            fsc = [fs_v[bl, pl.ds(16 * c, 16)] for c in range(8)]
            r1p, r2p = r1.at[p], r2.at[p]
            dv1p, dv2p = dv1[p], dv2[p]

            @pl.loop(0, CHUNK)
            def _(j):
                a1 = r1p[j, pl.ds(0, 16)] * ftc[0]
                a2 = r2p[j, pl.ds(0, 16)] * fsc[0]
                for c in range(1, 8):
                    a1 = a1 + r1p[j, pl.ds(16 * c, 16)] * ftc[c]
                    a2 = a2 + r2p[j, pl.ds(16 * c, 16)] * fsc[c]
                dv1p[pl.ds(j * 16, 16)] = a1
                dv2p[pl.ds(j * 16, 16)] = a2

            gsl = pl.ds((base + ci * CHUNK) * 16, CHUNK * 16)

            @pl.when(ci >= 2)
            def _():
                pltpu.make_async_copy(dv1p, da_hbm.at[gsl], ssem1[p]).wait()
                pltpu.make_async_copy(dv2p, db_hbm.at[gsl], ssem2[p]).wait()

            pltpu.async_copy(dv1p, da_hbm.at[gsl], ssem1[p])
            pltpu.async_copy(dv2p, db_hbm.at[gsl], ssem2[p])

            @pl.when(ci + 2 < CPW)
            def _():
                start_gathers(ci + 2, p)

        start_gathers(0, 0)
        start_gathers(1, 1)

        @pl.loop(0, CPW, step=2)
        def _(i):
            process(i, 0)
            process(i + 1, 1)

        gsl0 = pl.ds(base * 16, CHUNK * 16)
        for p in range(2):
            pltpu.make_async_copy(dv1[p], da_hbm.at[gsl0], ssem1[p]).wait()
            pltpu.make_async_copy(dv2[p], db_hbm.at[gsl0], ssem2[p]).wait()

    cp = pltpu.CompilerParams()
    if "needs_layout_passes" in pltpu.CompilerParams.__dataclass_fields__:
        cp = dataclasses.replace(cp, needs_layout_passes=False)
    k = pl.kernel(
        body,
        out_type=(out, out),
        mesh=mesh,
        compiler_params=cp,
        scratch_types=[
            pltpu.VMEM((PER_W,), jnp.int32),
            pltpu.VMEM((BPW, FEAT), jnp.float32),
            pltpu.VMEM((BPW, FEAT), jnp.float32),
            pltpu.VMEM((2, CHUNK, FEAT), jnp.float32),
            pltpu.VMEM((2, CHUNK, FEAT), jnp.float32),
            pltpu.VMEM((CHUNK * 16,), jnp.float32),
            pltpu.VMEM((CHUNK * 16,), jnp.float32),
            pltpu.VMEM((CHUNK * 16,), jnp.float32),
            pltpu.VMEM((CHUNK * 16,), jnp.float32),
        ] + [pltpu.SemaphoreType.DMA] * 8,
    )
    return k(memory_v1, memory_v2, flat_idx, f_t, f_s)


def _tc_tail(pa, pb):
    xa = pa.reshape(TOTAL // 8, 128)
    xb = pb.reshape(TOTAL // 8, 128)
    blk = 4096
    nblk = TOTAL // 8 // blk
    zscale = float(N_ROWS) / float(TOTAL)
    mpn = (KP1 - 1) / N_ROWS
    c = mpn + EPS
    log_mpn = math.log(mpn)
    nall = blk * 8
    npos = blk // 64
    ngarb = blk * 120

    li = np.arange(128)[:, None]
    lj = np.arange(128)[None, :]
    seg_mat = jnp.asarray((lj == (li // 16) * 16) * 1.0, jnp.float32)

    def body(xa_ref, xb_ref, m_ref, out_ref, acc):
        ph = pl.program_id(0)
        rows = lax.broadcasted_iota(jnp.int32, (blk, 128), 0)
        lanes = lax.broadcasted_iota(jnp.int32, (blk, 128), 1)
        seg = lanes % 16 == 0
        posm = (lanes == 0) & (rows % 64 == 0)
        dn = (((1,), (0,)), ((), ()))

        invt = jnp.float32(1.0 / NCE_T)
        da = lax.dot_general(xa_ref[...], m_ref[...], dn,
                             preferred_element_type=jnp.float32) * invt
        db = lax.dot_general(xb_ref[...], m_ref[...], dn,
                             preferred_element_type=jnp.float32) * invt
        ea = jnp.exp(da)
        eb = jnp.exp(db)

        @pl.when((ph == 0) & (pl.program_id(1) == 0))
        def _():
            acc[0] = 0.0
            acc[1] = 0.0
            out_ref[...] = jnp.zeros((1, 1), jnp.float32)

        @pl.when(ph == 0)
        def _():
            acc[0] += jnp.sum(ea) - float(ngarb)
            acc[1] += jnp.sum(eb) - float(ngarb)

        @pl.when(ph == 1)
        def _():
            zero = jnp.float32(0.0)
            ninf = jnp.float32(-jnp.inf)

            def side(d, e, z):
                o = e / z
                lgt = jnp.log(o + c)
                lg0 = log_mpn - lgt
                lg1 = jnp.where(o == 0.0, ninf, d - jnp.log(z)) - lgt
                return (jnp.sum(jnp.where(posm, lg1, zero))
                        + jnp.sum(jnp.where(seg, lg0, zero))
                        - jnp.sum(jnp.where(posm, lg0, zero)))

            la = side(da, ea, acc[0] * zscale)
            lb = side(db, eb, acc[1] * zscale)
            out_ref[...] += (-(la + lb) / BATCH).reshape(1, 1)

    return pl.pallas_call(
        body,
        grid=(2, nblk),
        in_specs=[pl.BlockSpec((blk, 128), lambda ph, j: (j, 0)),
                  pl.BlockSpec((blk, 128), lambda ph, j: (j, 0)),
                  pl.BlockSpec((128, 128), lambda ph, j: (0, 0))],
        out_specs=pl.BlockSpec((1, 1), lambda ph, j: (0, 0)),
        out_shape=jax.ShapeDtypeStruct((1, 1), jnp.float32),
        scratch_shapes=[pltpu.SMEM((2,), jnp.float32)],
    )(xa, xb, seg_mat)


def kernel(f_s, f_t, idx, contrast_idx, memory_v1, memory_v2):
    flat_idx = contrast_idx.reshape(-1)
    pa, pb = _sc_gather_dot(memory_v1, memory_v2, flat_idx, f_t, f_s)
    loss = _tc_tail(pa, pb)
    return loss.reshape(1)


# speedup vs baseline: 1.0781x; 1.0246x over previous
import dataclasses
import math

import jax
import jax.numpy as jnp
import numpy as np
from jax import lax
from jax.experimental import pallas as pl
from jax.experimental.pallas import tpu as pltpu
from jax.experimental.pallas import tpu_sc as plsc

FEAT = 128
N_ROWS = 1000000
BATCH = 1024
KP1 = 512
NCE_T = 0.07
EPS = 1e-7

NC, NS = 2, 16
NW = NC * NS
TOTAL = BATCH * KP1
PER_W = TOTAL // NW
CHUNK = 128
CPW = PER_W // CHUNK
BPW = BATCH // NW
CPB = KP1 // CHUNK


def _sc_gather_dot(memory_v1, memory_v2, flat_idx, f_t, f_s):
    mesh = plsc.VectorSubcoreMesh(core_axis_name="c", subcore_axis_name="s")
    out = jax.ShapeDtypeStruct((TOTAL * 16,), jnp.float32)

    def body(m1_hbm, m2_hbm, idx_hbm, ft_hbm, fs_hbm, da_hbm, db_hbm,
             idx_v, ft_v, fs_v, r1, r2, dv1a, dv1b, dv2a, dv2b,
             g1a, g1b, g2a, g2b, s1a, s1b, s2a, s2b):
        dv1 = (dv1a, dv1b)
        dv2 = (dv2a, dv2b)
        wid = lax.axis_index("s") * NC + lax.axis_index("c")
        base = wid * PER_W
        gsem1 = (g1a, g1b)
        gsem2 = (g2a, g2b)
        ssem1 = (s1a, s1b)
        ssem2 = (s2a, s2b)

        pltpu.sync_copy(idx_hbm.at[pl.ds(base, PER_W)], idx_v)
        pltpu.sync_copy(ft_hbm.at[pl.ds(wid * BPW, BPW)], ft_v)
        pltpu.sync_copy(fs_hbm.at[pl.ds(wid * BPW, BPW)], fs_v)

        def start_gathers(ci, p):
            sl = pl.ds(ci * CHUNK, CHUNK)
            pltpu.async_copy(m1_hbm.at[idx_v.at[sl]], r1.at[p], gsem1[p])
            pltpu.async_copy(m2_hbm.at[idx_v.at[sl]], r2.at[p], gsem2[p])

        def process(ci, p):
            sl = pl.ds(ci * CHUNK, CHUNK)
            pltpu.make_async_copy(m1_hbm.at[idx_v.at[sl]], r1.at[p],
                                  gsem1[p]).wait()
            pltpu.make_async_copy(m2_hbm.at[idx_v.at[sl]], r2.at[p],
                                  gsem2[p]).wait()

            bl = ci // CPB
            ftc = [ft_v[bl, pl.ds(16 * c, 16)] for c in range(8)]
            fsc = [fs_v[bl, pl.ds(16 * c, 16)] for c in range(8)]
            r1p, r2p = r1.at[p], r2.at[p]
            dv1p, dv2p = dv1[p], dv2[p]

            @pl.loop(0, CHUNK)
            def _(j):
                a1 = r1p[j, pl.ds(0, 16)] * ftc[0]
                a2 = r2p[j, pl.ds(0, 16)] * fsc[0]
                for c in range(1, 8):
                    a1 = a1 + r1p[j, pl.ds(16 * c, 16)] * ftc[c]
                    a2 = a2 + r2p[j, pl.ds(16 * c, 16)] * fsc[c]
                dv1p[pl.ds(j * 16, 16)] = a1
                dv2p[pl.ds(j * 16, 16)] = a2

            gsl = pl.ds((base + ci * CHUNK) * 16, CHUNK * 16)

            @pl.when(ci >= 2)
            def _():
                pltpu.make_async_copy(dv1p, da_hbm.at[gsl], ssem1[p]).wait()
                pltpu.make_async_copy(dv2p, db_hbm.at[gsl], ssem2[p]).wait()

            pltpu.async_copy(dv1p, da_hbm.at[gsl], ssem1[p])
            pltpu.async_copy(dv2p, db_hbm.at[gsl], ssem2[p])

            @pl.when(ci + 2 < CPW)
            def _():
                start_gathers(ci + 2, p)

        start_gathers(0, 0)
        start_gathers(1, 1)

        @pl.loop(0, CPW, step=2)
        def _(i):
            process(i, 0)
            process(i + 1, 1)

        gsl0 = pl.ds(base * 16, CHUNK * 16)
        for p in range(2):
            pltpu.make_async_copy(dv1[p], da_hbm.at[gsl0], ssem1[p]).wait()
            pltpu.make_async_copy(dv2[p], db_hbm.at[gsl0], ssem2[p]).wait()

    cp = pltpu.CompilerParams()
    if "needs_layout_passes" in pltpu.CompilerParams.__dataclass_fields__:
        cp = dataclasses.replace(cp, needs_layout_passes=False)
    k = pl.kernel(
        body,
        out_type=(out, out),
        mesh=mesh,
        compiler_params=cp,
        scratch_types=[
            pltpu.VMEM((PER_W,), jnp.int32),
            pltpu.VMEM((BPW, FEAT), jnp.float32),
            pltpu.VMEM((BPW, FEAT), jnp.float32),
            pltpu.VMEM((2, CHUNK, FEAT), jnp.float32),
            pltpu.VMEM((2, CHUNK, FEAT), jnp.float32),
            pltpu.VMEM((CHUNK * 16,), jnp.float32),
            pltpu.VMEM((CHUNK * 16,), jnp.float32),
            pltpu.VMEM((CHUNK * 16,), jnp.float32),
            pltpu.VMEM((CHUNK * 16,), jnp.float32),
        ] + [pltpu.SemaphoreType.DMA] * 8,
    )
    return k(memory_v1, memory_v2, flat_idx, f_t, f_s)


def _tc_tail(pa, pb):
    xa = pa.reshape(TOTAL // 8, 128)
    xb = pb.reshape(TOTAL // 8, 128)
    blk = 8192
    nblk = TOTAL // 8 // blk
    zscale = float(N_ROWS) / float(TOTAL)
    mpn = (KP1 - 1) / N_ROWS
    c = mpn + EPS
    log_mpn = math.log(mpn)
    nall = blk * 8
    npos = blk // 64
    ngarb = blk * 120

    li = np.arange(128)[:, None]
    lj = np.arange(128)[None, :]
    seg_mat = jnp.asarray((lj == (li // 16) * 16) * 1.0, jnp.float32)

    def body(xa_ref, xb_ref, m_ref, out_ref, acc):
        ph = pl.program_id(0)
        rows = lax.broadcasted_iota(jnp.int32, (blk, 128), 0)
        lanes = lax.broadcasted_iota(jnp.int32, (blk, 128), 1)
        seg = lanes % 16 == 0
        posm = (lanes == 0) & (rows % 64 == 0)
        dn = (((1,), (0,)), ((), ()))

        invt = jnp.float32(1.0 / NCE_T)
        da = lax.dot_general(xa_ref[...], m_ref[...], dn,
                             preferred_element_type=jnp.float32) * invt
        db = lax.dot_general(xb_ref[...], m_ref[...], dn,
                             preferred_element_type=jnp.float32) * invt
        ea = jnp.exp(da)
        eb = jnp.exp(db)

        @pl.when((ph == 0) & (pl.program_id(1) == 0))
        def _():
            acc[0] = 0.0
            acc[1] = 0.0
            out_ref[...] = jnp.zeros((1, 1), jnp.float32)

        @pl.when(ph == 0)
        def _():
            acc[0] += jnp.sum(ea) - float(ngarb)
            acc[1] += jnp.sum(eb) - float(ngarb)

        @pl.when(ph == 1)
        def _():
            zero = jnp.float32(0.0)
            ninf = jnp.float32(-jnp.inf)

            def side(d, e, z):
                o = e / z
                lgt = jnp.log(o + c)
                lg0 = log_mpn - lgt
                lg1 = jnp.where(o == 0.0, ninf, d - jnp.log(z)) - lgt
                return (jnp.sum(jnp.where(posm, lg1, zero))
                        + jnp.sum(jnp.where(seg, lg0, zero))
                        - jnp.sum(jnp.where(posm, lg0, zero)))

            la = side(da, ea, acc[0] * zscale)
            lb = side(db, eb, acc[1] * zscale)
            out_ref[...] += (-(la + lb) / BATCH).reshape(1, 1)

    return pl.pallas_call(
        body,
        grid=(2, nblk),
        in_specs=[pl.BlockSpec((blk, 128), lambda ph, j: (j, 0)),
                  pl.BlockSpec((blk, 128), lambda ph, j: (j, 0)),
                  pl.BlockSpec((128, 128), lambda ph, j: (0, 0))],
        out_specs=pl.BlockSpec((1, 1), lambda ph, j: (0, 0)),
        out_shape=jax.ShapeDtypeStruct((1, 1), jnp.float32),
        scratch_shapes=[pltpu.SMEM((2,), jnp.float32)],
    )(xa, xb, seg_mat)


def kernel(f_s, f_t, idx, contrast_idx, memory_v1, memory_v2):
    flat_idx = contrast_idx.reshape(-1)
    pa, pb = _sc_gather_dot(memory_v1, memory_v2, flat_idx, f_t, f_s)
    loss = _tc_tail(pa, pb)
    return loss.reshape(1)


# speedup vs baseline: 1.0782x; 1.0001x over previous
import dataclasses
import math

import jax
import jax.numpy as jnp
import numpy as np
from jax import lax
from jax.experimental import pallas as pl
from jax.experimental.pallas import tpu as pltpu
from jax.experimental.pallas import tpu_sc as plsc

FEAT = 128
N_ROWS = 1000000
BATCH = 1024
KP1 = 512
NCE_T = 0.07
EPS = 1e-7

NC, NS = 2, 16
NW = NC * NS
TOTAL = BATCH * KP1
PER_W = TOTAL // NW
CHUNK = 128
CPW = PER_W // CHUNK
BPW = BATCH // NW
CPB = KP1 // CHUNK


def _sc_gather_dot(memory_v1, memory_v2, flat_idx, f_t, f_s):
    mesh = plsc.VectorSubcoreMesh(core_axis_name="c", subcore_axis_name="s")
    out = jax.ShapeDtypeStruct((TOTAL * 16,), jnp.float32)

    def body(m1_hbm, m2_hbm, idx_hbm, ft_hbm, fs_hbm, da_hbm, db_hbm,
             idx_v, ft_v, fs_v, r1, r2, dv1a, dv1b, dv2a, dv2b,
             g1a, g1b, g2a, g2b, s1a, s1b, s2a, s2b):
        dv1 = (dv1a, dv1b)
        dv2 = (dv2a, dv2b)
        wid = lax.axis_index("s") * NC + lax.axis_index("c")
        base = wid * PER_W
        gsem1 = (g1a, g1b)
        gsem2 = (g2a, g2b)
        ssem1 = (s1a, s1b)
        ssem2 = (s2a, s2b)

        pltpu.sync_copy(idx_hbm.at[pl.ds(base, PER_W)], idx_v)
        pltpu.sync_copy(ft_hbm.at[pl.ds(wid * BPW, BPW)], ft_v)
        pltpu.sync_copy(fs_hbm.at[pl.ds(wid * BPW, BPW)], fs_v)

        def start_gathers(ci, p):
            sl = pl.ds(ci * CHUNK, CHUNK)
            pltpu.async_copy(m1_hbm.at[idx_v.at[sl]], r1.at[p], gsem1[p])
            pltpu.async_copy(m2_hbm.at[idx_v.at[sl]], r2.at[p], gsem2[p])

        def process(ci, p):
            sl = pl.ds(ci * CHUNK, CHUNK)
            pltpu.make_async_copy(m1_hbm.at[idx_v.at[sl]], r1.at[p],
                                  gsem1[p]).wait()
            pltpu.make_async_copy(m2_hbm.at[idx_v.at[sl]], r2.at[p],
                                  gsem2[p]).wait()

            bl = ci // CPB
            ftc = [ft_v[bl, pl.ds(16 * c, 16)] for c in range(8)]
            fsc = [fs_v[bl, pl.ds(16 * c, 16)] for c in range(8)]
            r1p, r2p = r1.at[p], r2.at[p]
            dv1p, dv2p = dv1[p], dv2[p]

            @pl.loop(0, CHUNK)
            def _(j):
                a1 = r1p[j, pl.ds(0, 16)] * ftc[0]
                a2 = r2p[j, pl.ds(0, 16)] * fsc[0]
                for c in range(1, 8):
                    a1 = a1 + r1p[j, pl.ds(16 * c, 16)] * ftc[c]
                    a2 = a2 + r2p[j, pl.ds(16 * c, 16)] * fsc[c]
                dv1p[pl.ds(j * 16, 16)] = a1
                dv2p[pl.ds(j * 16, 16)] = a2

            gsl = pl.ds((base + ci * CHUNK) * 16, CHUNK * 16)

            @pl.when(ci >= 2)
            def _():
                pltpu.make_async_copy(dv1p, da_hbm.at[gsl], ssem1[p]).wait()
                pltpu.make_async_copy(dv2p, db_hbm.at[gsl], ssem2[p]).wait()

            pltpu.async_copy(dv1p, da_hbm.at[gsl], ssem1[p])
            pltpu.async_copy(dv2p, db_hbm.at[gsl], ssem2[p])

            @pl.when(ci + 2 < CPW)
            def _():
                start_gathers(ci + 2, p)

        start_gathers(0, 0)
        start_gathers(1, 1)

        @pl.loop(0, CPW, step=2)
        def _(i):
            process(i, 0)
            process(i + 1, 1)

        gsl0 = pl.ds(base * 16, CHUNK * 16)
        for p in range(2):
            pltpu.make_async_copy(dv1[p], da_hbm.at[gsl0], ssem1[p]).wait()
            pltpu.make_async_copy(dv2[p], db_hbm.at[gsl0], ssem2[p]).wait()

    cp = pltpu.CompilerParams()
    if "needs_layout_passes" in pltpu.CompilerParams.__dataclass_fields__:
        cp = dataclasses.replace(cp, needs_layout_passes=False)
    k = pl.kernel(
        body,
        out_type=(out, out),
        mesh=mesh,
        compiler_params=cp,
        scratch_types=[
            pltpu.VMEM((PER_W,), jnp.int32),
            pltpu.VMEM((BPW, FEAT), jnp.float32),
            pltpu.VMEM((BPW, FEAT), jnp.float32),
            pltpu.VMEM((2, CHUNK, FEAT), jnp.float32),
            pltpu.VMEM((2, CHUNK, FEAT), jnp.float32),
            pltpu.VMEM((CHUNK * 16,), jnp.float32),
            pltpu.VMEM((CHUNK * 16,), jnp.float32),
            pltpu.VMEM((CHUNK * 16,), jnp.float32),
            pltpu.VMEM((CHUNK * 16,), jnp.float32),
        ] + [pltpu.SemaphoreType.DMA] * 8,
    )
    return k(memory_v1, memory_v2, flat_idx, f_t, f_s)


def _tc_tail(pa, pb):
    xa = pa.reshape(TOTAL // 8, 128)
    xb = pb.reshape(TOTAL // 8, 128)
    blk = 16384
    nblk = TOTAL // 8 // blk
    zscale = float(N_ROWS) / float(TOTAL)
    mpn = (KP1 - 1) / N_ROWS
    c = mpn + EPS
    log_mpn = math.log(mpn)
    nall = blk * 8
    npos = blk // 64
    ngarb = blk * 120

    li = np.arange(128)[:, None]
    lj = np.arange(128)[None, :]
    seg_mat = jnp.asarray((lj == (li // 16) * 16) * 1.0, jnp.float32)

    def body(xa_ref, xb_ref, m_ref, out_ref, acc):
        ph = pl.program_id(0)
        rows = lax.broadcasted_iota(jnp.int32, (blk, 128), 0)
        lanes = lax.broadcasted_iota(jnp.int32, (blk, 128), 1)
        seg = lanes % 16 == 0
        posm = (lanes == 0) & (rows % 64 == 0)
        dn = (((1,), (0,)), ((), ()))

        invt = jnp.float32(1.0 / NCE_T)
        da = lax.dot_general(xa_ref[...], m_ref[...], dn,
                             preferred_element_type=jnp.float32) * invt
        db = lax.dot_general(xb_ref[...], m_ref[...], dn,
                             preferred_element_type=jnp.float32) * invt
        ea = jnp.exp(da)
        eb = jnp.exp(db)

        @pl.when((ph == 0) & (pl.program_id(1) == 0))
        def _():
            acc[0] = 0.0
            acc[1] = 0.0
            out_ref[...] = jnp.zeros((1, 1), jnp.float32)

        @pl.when(ph == 0)
        def _():
            acc[0] += jnp.sum(ea) - float(ngarb)
            acc[1] += jnp.sum(eb) - float(ngarb)

        @pl.when(ph == 1)
        def _():
            zero = jnp.float32(0.0)
            ninf = jnp.float32(-jnp.inf)

            def side(d, e, z):
                o = e / z
                lgt = jnp.log(o + c)
                lg0 = log_mpn - lgt
                lg1 = jnp.where(o == 0.0, ninf, d - jnp.log(z)) - lgt
                return (jnp.sum(jnp.where(posm, lg1, zero))
                        + jnp.sum(jnp.where(seg, lg0, zero))
                        - jnp.sum(jnp.where(posm, lg0, zero)))

            la = side(da, ea, acc[0] * zscale)
            lb = side(db, eb, acc[1] * zscale)
            out_ref[...] += (-(la + lb) / BATCH).reshape(1, 1)

    return pl.pallas_call(
        body,
        grid=(2, nblk),
        in_specs=[pl.BlockSpec((blk, 128), lambda ph, j: (j, 0)),
                  pl.BlockSpec((blk, 128), lambda ph, j: (j, 0)),
                  pl.BlockSpec((128, 128), lambda ph, j: (0, 0))],
        out_specs=pl.BlockSpec((1, 1), lambda ph, j: (0, 0)),
        out_shape=jax.ShapeDtypeStruct((1, 1), jnp.float32),
        scratch_shapes=[pltpu.SMEM((2,), jnp.float32)],
    )(xa, xb, seg_mat)


def kernel(f_s, f_t, idx, contrast_idx, memory_v1, memory_v2):
    flat_idx = contrast_idx.reshape(-1)
    pa, pb = _sc_gather_dot(memory_v1, memory_v2, flat_idx, f_t, f_s)
    loss = _tc_tail(pa, pb)
    return loss.reshape(1)
